# Initial kernel scaffold; baseline (speedup 1.0000x reference)
import functools

import jax
import jax.numpy as jnp
from jax import lax
from jax.experimental import pallas as pl
from jax.experimental.pallas import tpu as pltpu
from jax.experimental.pallas import tpu_sc as plsc

N_USERS = 5000
N_NODES = 10000
NPAD = 10240
DH = 64
NE = 320000
CHUNK = 128
NCH = 157
EPAD = 16 * NCH * CHUNK
NSL = NPAD // 16
DUMMY = N_NODES
NLAYERS = 3

_mesh = plsc.VectorSubcoreMesh(
    core_axis_name="c", subcore_axis_name="s", num_cores=2, num_subcores=16
)


@functools.partial(
    pl.kernel,
    out_type=[
        jax.ShapeDtypeStruct((2, NPAD, DH), jnp.float32),
        jax.ShapeDtypeStruct((2 * NPAD, DH), jnp.float32),
    ],
    mesh=_mesh,
    scratch_types=[
        pltpu.VMEM((NCH, CHUNK), jnp.int32),
        pltpu.VMEM((NCH, CHUNK), jnp.int32),
        pltpu.VMEM((CHUNK, DH), jnp.float32),
        pltpu.VMEM((CHUNK, DH), jnp.float32),
        pltpu.VMEM((NSL, DH), jnp.float32),
        pltpu.VMEM((NSL, 16), jnp.float32),
        pltpu.VMEM((CHUNK, 16), jnp.float32),
        pltpu.VMEM((CHUNK, DH), jnp.float32),
        pltpu.VMEM_SHARED((NPAD, DH), jnp.float32),
        pltpu.VMEM_SHARED((NPAD, 16), jnp.float32),
        pltpu.SemaphoreType.DMA,
    ],
)
def _lightgcn_sc(xin, rows_h, cols_h, out, ybuf,
                 rows_v, cols_v, gbuf, wb, sumb, dv, onesb, zerob,
                 acc, degs, sem):
    cid = lax.axis_index("c")
    sid = lax.axis_index("s")
    base_n = sid * NSL
    xoff = cid * NPAD + base_n

    pltpu.sync_copy(rows_h.at[sid], rows_v)
    pltpu.sync_copy(cols_h.at[sid], cols_v)
    off = (cid * NPAD).astype(jnp.int32)

    def _shift_rows(j, _):
        for k in range(CHUNK // 16):
            sl = pl.ds(k * 16, 16)
            rows_v[j, sl] = rows_v[j, sl] + off
        return 0
    lax.fori_loop(0, NCH, _shift_rows, 0)

    def _fill_ones(j, _):
        onesb[j, :] = jnp.full((16,), 1.0, jnp.float32)
        return 0
    lax.fori_loop(0, CHUNK, _fill_ones, 0)

    def _fill_zero(j, _):
        for k in range(DH // 16):
            zerob[j, pl.ds(k * 16, 16)] = jnp.zeros((16,), jnp.float32)
        return 0
    lax.fori_loop(0, CHUNK, _fill_zero, 0)

    def _zero_dv(n, _):
        dv[n, :] = jnp.zeros((16,), jnp.float32)
        return 0
    lax.fori_loop(0, NSL, _zero_dv, 0)
    pltpu.sync_copy(dv, degs.at[pl.ds(base_n, NSL)])
    for t in range(NSL // CHUNK):
        pltpu.sync_copy(zerob, acc.at[pl.ds(base_n + t * CHUNK, CHUNK)])
    plsc.subcore_barrier()

    def _deg_chunk(j, _):
        pltpu.sync_copy(onesb, degs.at[cols_v.at[j]], add=True)
        return 0
    lax.fori_loop(0, NCH, _deg_chunk, 0)
    plsc.subcore_barrier()

    pltpu.sync_copy(degs.at[pl.ds(base_n, NSL)], dv)

    def _newton(n, _):
        d = dv[n, :]
        i = plsc.bitcast(d, jnp.int32)
        y = plsc.bitcast(jnp.full((16,), 0x5F3759DF, jnp.int32) - (i >> 1),
                         jnp.float32)
        for _ in range(3):
            y = y * (1.5 - 0.5 * d * y * y)
        dv[n, :] = jnp.where(d > 0.5, y, jnp.zeros((16,), jnp.float32))
        return 0
    lax.fori_loop(0, NSL, _newton, 0)

    for t in range(NSL // CHUNK):
        pltpu.sync_copy(xin.at[pl.ds(xoff + t * CHUNK, CHUNK)], wb)

        def _y0(m, _, t=t):
            n = t * CHUNK + m
            b = dv[n, :]
            for k in range(DH // 16):
                sl = pl.ds(k * 16, 16)
                a = wb[m, sl]
                sumb[n, sl] = a
                wb[m, sl] = a * b
            return 0
        lax.fori_loop(0, CHUNK, _y0, 0)
        pltpu.sync_copy(wb, ybuf.at[pl.ds(xoff + t * CHUNK, CHUNK)])
    plsc.subcore_barrier()

    for layer in range(NLAYERS):
        last = layer == NLAYERS - 1

        def _edge_chunk(j, _):
            pltpu.async_copy(ybuf.at[rows_v.at[j]], gbuf, sem).wait()
            pltpu.sync_copy(gbuf, acc.at[cols_v.at[j]], add=True)
            return 0
        lax.fori_loop(0, NCH, _edge_chunk, 0)
        plsc.subcore_barrier()

        for t in range(NSL // CHUNK):
            sl_nodes = pl.ds(base_n + t * CHUNK, CHUNK)
            pltpu.sync_copy(acc.at[sl_nodes], wb)
            if not last:
                pltpu.sync_copy(zerob, acc.at[sl_nodes])

            def _nodes(m, _, t=t, last=last):
                n = t * CHUNK + m
                b = dv[n, :]
                for k in range(DH // 16):
                    sl = pl.ds(k * 16, 16)
                    a = wb[m, sl] * b
                    s = sumb[n, sl] + a
                    sumb[n, sl] = s
                    if last:
                        wb[m, sl] = s * (1.0 / (NLAYERS + 1))
                    else:
                        wb[m, sl] = a * b
                return 0
            lax.fori_loop(0, CHUNK, _nodes, 0)
            if last:
                pltpu.sync_copy(wb, out.at[cid, pl.ds(base_n + t * CHUNK, CHUNK)])
            else:
                pltpu.sync_copy(wb, ybuf.at[pl.ds(xoff + t * CHUNK, CHUNK)])
        plsc.subcore_barrier()


@jax.jit
def kernel(user_emb, item_emb, edge_index):
    x = jnp.concatenate([user_emb, item_emb], axis=0)
    xpad = jnp.pad(x, ((0, NPAD - N_NODES), (0, 0)))
    xin = jnp.concatenate([xpad[:, :DH], xpad[:, DH:]], axis=0)

    rows = edge_index[0].astype(jnp.int32)
    cols = edge_index[1].astype(jnp.int32)
    rows = jnp.pad(rows, (0, EPAD - NE), constant_values=DUMMY)
    cols = jnp.pad(cols, (0, EPAD - NE), constant_values=DUMMY)
    rows_h = rows.reshape(16, NCH, CHUNK)
    cols_h = cols.reshape(16, NCH, CHUNK)

    out, _y = _lightgcn_sc(xin, rows_h, cols_h)
    final = jnp.concatenate([out[0, :N_NODES], out[1, :N_NODES]], axis=1)
    return final[:N_USERS], final[N_USERS:]


# speedup vs baseline: 6.1286x; 6.1286x over previous
import functools

import jax
import jax.numpy as jnp
from jax import lax
from jax.experimental import pallas as pl
from jax.experimental.pallas import tpu as pltpu
from jax.experimental.pallas import tpu_sc as plsc

N_USERS = 5000
N_NODES = 10000
NPAD = 10240
DH = 64
NE = 320000
CHUNK = 128
NG = 10
GSZ = 16
EPAD = 16 * NG * GSZ * CHUNK
NSL = NPAD // 16
DUMMY = N_NODES
NLAYERS = 3

_mesh = plsc.VectorSubcoreMesh(
    core_axis_name="c", subcore_axis_name="s", num_cores=2, num_subcores=16
)


@functools.partial(
    pl.kernel,
    out_type=[
        jax.ShapeDtypeStruct((2, NPAD, DH), jnp.float32),
        jax.ShapeDtypeStruct((2 * NPAD, DH), jnp.float32),
    ],
    mesh=_mesh,
    scratch_types=[
        pltpu.VMEM((GSZ, CHUNK), jnp.int32),
        pltpu.VMEM((GSZ, CHUNK), jnp.int32),
        pltpu.VMEM((CHUNK, DH), jnp.float32),
        pltpu.VMEM((CHUNK, DH), jnp.float32),
        pltpu.VMEM((NSL, DH), jnp.float32),
        pltpu.VMEM((NSL, 16), jnp.float32),
        pltpu.VMEM((CHUNK, 16), jnp.float32),
        pltpu.VMEM_SHARED((NPAD, DH), jnp.float32),
        pltpu.VMEM_SHARED((NPAD, 16), jnp.float32),
        pltpu.SemaphoreType.DMA,
    ],
    compiler_params=pltpu.CompilerParams(use_tc_tiling_on_sc=False),
)
def _lightgcn_sc(xin, rows_h, cols_h, out, ybuf,
                 rowsb, colsb, gbuf, wb, sumb, dv, onesb,
                 acc, degs, sem):
    cid = lax.axis_index("c")
    sid = lax.axis_index("s")
    base_n = sid * NSL
    xoff = cid * NPAD + base_n
    off = (cid * NPAD).astype(jnp.int32)

    def _fill_ones(j, _):
        onesb[j, :] = jnp.full((16,), 1.0, jnp.float32)
        return 0
    lax.fori_loop(0, CHUNK, _fill_ones, 0)

    def _zero_wb(j, _):
        for k in range(DH // 16):
            wb[j, pl.ds(k * 16, 16)] = jnp.zeros((16,), jnp.float32)
        return 0

    def _zero_dv(n, _):
        dv[n, :] = jnp.zeros((16,), jnp.float32)
        return 0
    lax.fori_loop(0, NSL, _zero_dv, 0)
    pltpu.sync_copy(dv, degs.at[pl.ds(base_n, NSL)])
    lax.fori_loop(0, CHUNK, _zero_wb, 0)
    for t in range(NSL // CHUNK):
        pltpu.sync_copy(wb, acc.at[pl.ds(base_n + t * CHUNK, CHUNK)])
    plsc.subcore_barrier()

    def _deg_group(g, _):
        pltpu.sync_copy(cols_h.at[sid, g], colsb)

        def _deg_chunk(j, _):
            pltpu.sync_copy(onesb, degs.at[colsb.at[j]], add=True)
            return 0
        lax.fori_loop(0, GSZ, _deg_chunk, 0)
        return 0
    lax.fori_loop(0, NG, _deg_group, 0)
    plsc.subcore_barrier()

    pltpu.sync_copy(degs.at[pl.ds(base_n, NSL)], dv)

    def _newton(n, _):
        d = dv[n, :]
        i = lax.bitcast_convert_type(d, jnp.int32)
        y = lax.bitcast_convert_type(
            jnp.full((16,), 0x5F3759DF, jnp.int32) - (i >> 1), jnp.float32)
        for _ in range(3):
            y = y * (1.5 - 0.5 * d * y * y)
        dv[n, :] = jnp.where(d > 0.5, y, jnp.zeros((16,), jnp.float32))
        return 0
    lax.fori_loop(0, NSL, _newton, 0)

    for t in range(NSL // CHUNK):
        pltpu.sync_copy(xin.at[pl.ds(xoff + t * CHUNK, CHUNK)], wb)

        def _y0(m, _, t=t):
            n = t * CHUNK + m
            b = dv[n, :]
            for k in range(DH // 16):
                sl = pl.ds(k * 16, 16)
                a = wb[m, sl]
                sumb[n, sl] = a
                wb[m, sl] = a * b
            return 0
        lax.fori_loop(0, CHUNK, _y0, 0)
        pltpu.sync_copy(wb, ybuf.at[pl.ds(xoff + t * CHUNK, CHUNK)])
    plsc.subcore_barrier()

    for layer in range(NLAYERS):
        last = layer == NLAYERS - 1

        def _edge_group(g, _):
            pltpu.sync_copy(rows_h.at[sid, g], rowsb)
            pltpu.sync_copy(cols_h.at[sid, g], colsb)

            def _shift(j, _):
                for k in range(CHUNK // 16):
                    sl = pl.ds(k * 16, 16)
                    rowsb[j, sl] = rowsb[j, sl] + off
                return 0
            lax.fori_loop(0, GSZ, _shift, 0)

            def _edge_chunk(j, _):
                pltpu.async_copy(ybuf.at[rowsb.at[j]], gbuf, sem).wait()
                pltpu.sync_copy(gbuf, acc.at[colsb.at[j]], add=True)
                return 0
            lax.fori_loop(0, GSZ, _edge_chunk, 0)
            return 0
        lax.fori_loop(0, NG, _edge_group, 0)
        plsc.subcore_barrier()

        for t in range(NSL // CHUNK):
            sl_nodes = pl.ds(base_n + t * CHUNK, CHUNK)
            pltpu.sync_copy(acc.at[sl_nodes], wb)

            def _nodes(m, _, t=t, last=last):
                n = t * CHUNK + m
                b = dv[n, :]
                for k in range(DH // 16):
                    sl = pl.ds(k * 16, 16)
                    a = wb[m, sl] * b
                    s = sumb[n, sl] + a
                    sumb[n, sl] = s
                    if last:
                        wb[m, sl] = s * (1.0 / (NLAYERS + 1))
                    else:
                        wb[m, sl] = a * b
                return 0
            lax.fori_loop(0, CHUNK, _nodes, 0)
            if last:
                pltpu.sync_copy(wb, out.at[cid, pl.ds(base_n + t * CHUNK, CHUNK)])
            else:
                pltpu.sync_copy(wb, ybuf.at[pl.ds(xoff + t * CHUNK, CHUNK)])
                lax.fori_loop(0, CHUNK, _zero_wb, 0)
                pltpu.sync_copy(wb, acc.at[sl_nodes])
        plsc.subcore_barrier()


@jax.jit
def kernel(user_emb, item_emb, edge_index):
    x = jnp.concatenate([user_emb, item_emb], axis=0)
    xpad = jnp.pad(x, ((0, NPAD - N_NODES), (0, 0)))
    xin = jnp.concatenate([xpad[:, :DH], xpad[:, DH:]], axis=0)

    rows = edge_index[0].astype(jnp.int32)
    cols = edge_index[1].astype(jnp.int32)
    rows = jnp.pad(rows, (0, EPAD - NE), constant_values=DUMMY)
    cols = jnp.pad(cols, (0, EPAD - NE), constant_values=DUMMY)
    rows_h = rows.reshape(16, NG, GSZ, CHUNK)
    cols_h = cols.reshape(16, NG, GSZ, CHUNK)

    out, _y = _lightgcn_sc(xin, rows_h, cols_h)
    final = jnp.concatenate([out[0, :N_NODES], out[1, :N_NODES]], axis=1)
    return final[:N_USERS], final[N_USERS:]
